# baseline (speedup 1.0000x reference)
import functools

import jax
import jax.numpy as jnp
from jax import lax
from jax.experimental import pallas as pl
from jax.experimental.pallas import tpu as pltpu
from jax.experimental.pallas import tpu_sc as plsc

N = 10000
E = 320000
NC = 2
NS = 16
K = 80
NCH = 126
E_PAD = NC * NS * NCH * K
NJ = 16
RPT = 624
REM = N - NS * RPT
ZR = 48


def _sc_edge_scatter(D):
    mesh = plsc.VectorSubcoreMesh(core_axis_name="c", subcore_axis_name="s")

    def body(h, src, dst, out, sidx, didx, rows0, rows1, zbuf, acc,
             semi, sem0, sem1):
        cid = lax.axis_index("c")
        sid = lax.axis_index("s")

        ic = pltpu.async_copy(src.at[cid, sid], sidx, semi)
        dc = pltpu.async_copy(dst.at[cid, sid], didx, semi)

        def zrow(r, _):
            for j in range(D // 16):
                zbuf[r, pl.ds(j * 16, 16)] = jnp.zeros((16,), jnp.float32)
            return 0
        lax.fori_loop(0, ZR, zrow, 0)

        def zacc(j, _):
            pltpu.sync_copy(zbuf, acc.at[pl.ds(sid * RPT + j * ZR, ZR)])
            return 0
        lax.fori_loop(0, RPT // ZR, zacc, 0)

        @pl.when(sid == NS - 1)
        def _():
            pltpu.sync_copy(zbuf.at[pl.ds(0, REM)],
                            acc.at[pl.ds(NS * RPT, REM)])

        ic.wait()
        dc.wait()

        plsc.subcore_barrier()

        pltpu.async_copy(h.at[sidx.at[0]], rows0, sem0)
        pltpu.async_copy(h.at[sidx.at[1]], rows1, sem1)

        def step(j, _):
            c0 = 2 * j
            c1 = 2 * j + 1
            n0 = jnp.where(c0 + 2 < NCH, c0 + 2, 0)
            n1 = jnp.where(c1 + 2 < NCH, c1 + 2, 1)
            pltpu.make_async_copy(h.at[sidx.at[c0]], rows0, sem0).wait()
            pltpu.sync_copy(rows0, acc.at[didx.at[c0]], add=True)
            pltpu.async_copy(h.at[sidx.at[n0]], rows0, sem0)
            pltpu.make_async_copy(h.at[sidx.at[c1]], rows1, sem1).wait()
            pltpu.sync_copy(rows1, acc.at[didx.at[c1]], add=True)
            pltpu.async_copy(h.at[sidx.at[n1]], rows1, sem1)
            return 0
        lax.fori_loop(0, NCH // 2, step, 0)

        pltpu.make_async_copy(h.at[sidx.at[0]], rows0, sem0).wait()
        pltpu.make_async_copy(h.at[sidx.at[1]], rows1, sem1).wait()

        plsc.subcore_barrier()
        pltpu.sync_copy(acc.at[pl.ds(sid * RPT, RPT)],
                        out.at[cid, pl.ds(sid * RPT, RPT)])

        @pl.when(sid == NS - 1)
        def _():
            pltpu.sync_copy(acc.at[pl.ds(NS * RPT, REM)],
                            out.at[cid, pl.ds(NS * RPT, REM)])

    return pl.kernel(
        body,
        out_type=jax.ShapeDtypeStruct((NC, N, D), jnp.float32),
        mesh=mesh,
        scratch_types=[
            pltpu.VMEM((NCH, K), jnp.int32),
            pltpu.VMEM((NCH, K), jnp.int32),
            pltpu.VMEM((K, D), jnp.float32),
            pltpu.VMEM((K, D), jnp.float32),
            pltpu.VMEM((ZR, D), jnp.float32),
            pltpu.VMEM_SHARED((N + NJ, D), jnp.float32),
            pltpu.SemaphoreType.DMA,
            pltpu.SemaphoreType.DMA,
            pltpu.SemaphoreType.DMA,
        ],
        compiler_params=pltpu.CompilerParams(use_tc_tiling_on_sc=False),
    )


def _mm_body(x_ref, w_ref, o_ref):
    o_ref[...] = jnp.dot(x_ref[...], w_ref[...],
                         preferred_element_type=jnp.float32)


def _relu_mm_body(p_ref, w_ref, o_ref):
    r = jnp.maximum(p_ref[0] + p_ref[1], 0.0)
    o_ref[...] = jnp.dot(r, w_ref[...], preferred_element_type=jnp.float32)


def _add_body(q_ref, o_ref):
    o_ref[...] = q_ref[0] + q_ref[1]


@functools.lru_cache(maxsize=None)
def _layers():
    return _sc_edge_scatter(128), _sc_edge_scatter(64)


def kernel(x, adj, W1, W2):
    pad = E_PAD - E
    src = jnp.concatenate(
        [adj[0].astype(jnp.int32), jnp.zeros((pad,), jnp.int32)]
    ).reshape(NC, NS, NCH, K)
    dst = jnp.concatenate(
        [adj[1].astype(jnp.int32),
         N + (jnp.arange(pad, dtype=jnp.int32) % NJ)]
    ).reshape(NC, NS, NCH, K)
    sc1, sc2 = _layers()

    h1 = pl.pallas_call(
        _mm_body,
        out_shape=jax.ShapeDtypeStruct((N, 128), jnp.float32),
    )(x, W1)
    p1 = sc1(h1, src, dst)
    h2 = pl.pallas_call(
        _relu_mm_body,
        out_shape=jax.ShapeDtypeStruct((N, 64), jnp.float32),
    )(p1, W2)
    p2 = sc2(h2, src, dst)
    out = pl.pallas_call(
        _add_body,
        out_shape=jax.ShapeDtypeStruct((N, 64), jnp.float32),
    )(p2)
    return out


# speedup vs baseline: 1.0532x; 1.0532x over previous
import functools

import jax
import jax.numpy as jnp
from jax import lax
from jax.experimental import pallas as pl
from jax.experimental.pallas import tpu as pltpu
from jax.experimental.pallas import tpu_sc as plsc

N = 10000
E = 320000
NC = 2
NS = 16
K = 80
NCH = 125
RPT = 624
REM = N - NS * RPT
ZR = 48


def _sc_edge_scatter(D):
    mesh = plsc.VectorSubcoreMesh(core_axis_name="c", subcore_axis_name="s")

    def body(h, src, dst, out, sidx, didx, rows, zbuf, acc, semi, sem):
        cid = lax.axis_index("c")
        sid = lax.axis_index("s")

        ic = pltpu.async_copy(src.at[cid, sid], sidx, semi)
        dc = pltpu.async_copy(dst.at[cid, sid], didx, semi)

        def zrow(r, _):
            for j in range(D // 16):
                zbuf[r, pl.ds(j * 16, 16)] = jnp.zeros((16,), jnp.float32)
            return 0
        lax.fori_loop(0, ZR, zrow, 0)

        def zacc(j, _):
            pltpu.sync_copy(zbuf, acc.at[pl.ds(sid * RPT + j * ZR, ZR)])
            return 0
        lax.fori_loop(0, RPT // ZR, zacc, 0)

        @pl.when(sid == NS - 1)
        def _():
            pltpu.sync_copy(zbuf.at[pl.ds(0, REM)],
                            acc.at[pl.ds(NS * RPT, REM)])

        ic.wait()
        dc.wait()

        plsc.subcore_barrier()

        def step(j, _):
            pltpu.async_copy(h.at[sidx.at[j]], rows, sem)
            pltpu.make_async_copy(h.at[sidx.at[j]], rows, sem).wait()
            pltpu.sync_copy(rows, acc.at[didx.at[j]], add=True)
            return 0
        lax.fori_loop(0, NCH, step, 0)

        plsc.subcore_barrier()
        pltpu.sync_copy(acc.at[pl.ds(sid * RPT, RPT)],
                        out.at[cid, pl.ds(sid * RPT, RPT)])

        @pl.when(sid == NS - 1)
        def _():
            pltpu.sync_copy(acc.at[pl.ds(NS * RPT, REM)],
                            out.at[cid, pl.ds(NS * RPT, REM)])

    return pl.kernel(
        body,
        out_type=jax.ShapeDtypeStruct((NC, N, D), jnp.float32),
        mesh=mesh,
        scratch_types=[
            pltpu.VMEM((NCH, K), jnp.int32),
            pltpu.VMEM((NCH, K), jnp.int32),
            pltpu.VMEM((K, D), jnp.float32),
            pltpu.VMEM((ZR, D), jnp.float32),
            pltpu.VMEM_SHARED((N, D), jnp.float32),
            pltpu.SemaphoreType.DMA,
            pltpu.SemaphoreType.DMA,
        ],
        compiler_params=pltpu.CompilerParams(use_tc_tiling_on_sc=False),
    )


def _mm_body(x_ref, w_ref, o_ref):
    o_ref[...] = jnp.dot(x_ref[...], w_ref[...],
                         preferred_element_type=jnp.float32)


def _relu_mm_body(p_ref, w_ref, o_ref):
    r = jnp.maximum(p_ref[0] + p_ref[1], 0.0)
    o_ref[...] = jnp.dot(r, w_ref[...], preferred_element_type=jnp.float32)


def _add_body(q_ref, o_ref):
    o_ref[...] = q_ref[0] + q_ref[1]


@functools.lru_cache(maxsize=None)
def _layers():
    return _sc_edge_scatter(128), _sc_edge_scatter(64)


def kernel(x, adj, W1, W2):
    src = adj[0].astype(jnp.int32).reshape(NC, NS, NCH, K)
    dst = adj[1].astype(jnp.int32).reshape(NC, NS, NCH, K)
    sc1, sc2 = _layers()

    h1 = pl.pallas_call(
        _mm_body,
        out_shape=jax.ShapeDtypeStruct((N, 128), jnp.float32),
    )(x, W1)
    p1 = sc1(h1, src, dst)
    h2 = pl.pallas_call(
        _relu_mm_body,
        out_shape=jax.ShapeDtypeStruct((N, 64), jnp.float32),
    )(p1, W2)
    p2 = sc2(h2, src, dst)
    out = pl.pallas_call(
        _add_body,
        out_shape=jax.ShapeDtypeStruct((N, 64), jnp.float32),
    )(p2)
    return out


# speedup vs baseline: 1.1275x; 1.0705x over previous
import functools

import jax
import jax.numpy as jnp
from jax import lax
from jax.experimental import pallas as pl
from jax.experimental.pallas import tpu as pltpu
from jax.experimental.pallas import tpu_sc as plsc

N = 10000
E = 320000
NC = 2
NS = 16
K = 100
NCH = 100
RPT = 624
REM = N - NS * RPT
ZR = 48


def _sc_edge_scatter(D):
    mesh = plsc.VectorSubcoreMesh(core_axis_name="c", subcore_axis_name="s")

    def body(h, src, dst, out, sidx, didx, rows, zbuf, acc, semi, sem):
        cid = lax.axis_index("c")
        sid = lax.axis_index("s")

        ic = pltpu.async_copy(src.at[cid, sid], sidx, semi)
        dc = pltpu.async_copy(dst.at[cid, sid], didx, semi)

        def zrow(r, _):
            for j in range(D // 16):
                zbuf[r, pl.ds(j * 16, 16)] = jnp.zeros((16,), jnp.float32)
            return 0
        lax.fori_loop(0, ZR, zrow, 0)

        def zacc(j, _):
            pltpu.sync_copy(zbuf, acc.at[pl.ds(sid * RPT + j * ZR, ZR)])
            return 0
        lax.fori_loop(0, RPT // ZR, zacc, 0)

        @pl.when(sid == NS - 1)
        def _():
            pltpu.sync_copy(zbuf.at[pl.ds(0, REM)],
                            acc.at[pl.ds(NS * RPT, REM)])

        ic.wait()
        dc.wait()

        plsc.subcore_barrier()

        def step(j, _):
            pltpu.async_copy(h.at[sidx.at[j]], rows, sem)
            pltpu.make_async_copy(h.at[sidx.at[j]], rows, sem).wait()
            pltpu.sync_copy(rows, acc.at[didx.at[j]], add=True)
            return 0
        lax.fori_loop(0, NCH, step, 0)

        plsc.subcore_barrier()
        pltpu.sync_copy(acc.at[pl.ds(sid * RPT, RPT)],
                        out.at[cid, pl.ds(sid * RPT, RPT)])

        @pl.when(sid == NS - 1)
        def _():
            pltpu.sync_copy(acc.at[pl.ds(NS * RPT, REM)],
                            out.at[cid, pl.ds(NS * RPT, REM)])

    return pl.kernel(
        body,
        out_type=jax.ShapeDtypeStruct((NC, N, D), jnp.float32),
        mesh=mesh,
        scratch_types=[
            pltpu.VMEM((NCH, K), jnp.int32),
            pltpu.VMEM((NCH, K), jnp.int32),
            pltpu.VMEM((K, D), jnp.float32),
            pltpu.VMEM((ZR, D), jnp.float32),
            pltpu.VMEM_SHARED((N, D), jnp.float32),
            pltpu.SemaphoreType.DMA,
            pltpu.SemaphoreType.DMA,
        ],
        compiler_params=pltpu.CompilerParams(use_tc_tiling_on_sc=False),
    )


def _mm_body(x_ref, w_ref, o_ref):
    o_ref[...] = jnp.dot(x_ref[...], w_ref[...],
                         preferred_element_type=jnp.float32)


def _relu_mm_body(p_ref, w_ref, o_ref):
    r = jnp.maximum(p_ref[0] + p_ref[1], 0.0)
    o_ref[...] = jnp.dot(r, w_ref[...], preferred_element_type=jnp.float32)


def _add_body(q_ref, o_ref):
    o_ref[...] = q_ref[0] + q_ref[1]


@functools.lru_cache(maxsize=None)
def _layers():
    return _sc_edge_scatter(128), _sc_edge_scatter(64)


def kernel(x, adj, W1, W2):
    src = adj[0].astype(jnp.int32).reshape(NC, NS, NCH, K)
    dst = adj[1].astype(jnp.int32).reshape(NC, NS, NCH, K)
    sc1, sc2 = _layers()

    h1 = pl.pallas_call(
        _mm_body,
        out_shape=jax.ShapeDtypeStruct((N, 128), jnp.float32),
    )(x, W1)
    p1 = sc1(h1, src, dst)
    h2 = pl.pallas_call(
        _relu_mm_body,
        out_shape=jax.ShapeDtypeStruct((N, 64), jnp.float32),
    )(p1, W2)
    p2 = sc2(h2, src, dst)
    out = pl.pallas_call(
        _add_body,
        out_shape=jax.ShapeDtypeStruct((N, 64), jnp.float32),
    )(p2)
    return out


# speedup vs baseline: 1.2139x; 1.0766x over previous
import functools

import jax
import jax.numpy as jnp
from jax import lax
from jax.experimental import pallas as pl
from jax.experimental.pallas import tpu as pltpu
from jax.experimental.pallas import tpu_sc as plsc

N = 10000
E = 320000
NC = 2
NS = 16
K = 125
NCH = 80
RPT = 624
REM = N - NS * RPT
ZR = 48


def _sc_edge_scatter(D):
    mesh = plsc.VectorSubcoreMesh(core_axis_name="c", subcore_axis_name="s")

    def body(h, src, dst, out, sidx, didx, rows, zbuf, acc, semi, sem):
        cid = lax.axis_index("c")
        sid = lax.axis_index("s")

        ic = pltpu.async_copy(src.at[cid, sid], sidx, semi)
        dc = pltpu.async_copy(dst.at[cid, sid], didx, semi)

        def zrow(r, _):
            for j in range(D // 16):
                zbuf[r, pl.ds(j * 16, 16)] = jnp.zeros((16,), jnp.float32)
            return 0
        lax.fori_loop(0, ZR, zrow, 0)

        def zacc(j, _):
            pltpu.sync_copy(zbuf, acc.at[pl.ds(sid * RPT + j * ZR, ZR)])
            return 0
        lax.fori_loop(0, RPT // ZR, zacc, 0)

        @pl.when(sid == NS - 1)
        def _():
            pltpu.sync_copy(zbuf.at[pl.ds(0, REM)],
                            acc.at[pl.ds(NS * RPT, REM)])

        ic.wait()
        dc.wait()

        plsc.subcore_barrier()

        def step(j, _):
            pltpu.async_copy(h.at[sidx.at[j]], rows, sem)
            pltpu.make_async_copy(h.at[sidx.at[j]], rows, sem).wait()
            pltpu.sync_copy(rows, acc.at[didx.at[j]], add=True)
            return 0
        lax.fori_loop(0, NCH, step, 0)

        plsc.subcore_barrier()
        pltpu.sync_copy(acc.at[pl.ds(sid * RPT, RPT)],
                        out.at[cid, pl.ds(sid * RPT, RPT)])

        @pl.when(sid == NS - 1)
        def _():
            pltpu.sync_copy(acc.at[pl.ds(NS * RPT, REM)],
                            out.at[cid, pl.ds(NS * RPT, REM)])

    return pl.kernel(
        body,
        out_type=jax.ShapeDtypeStruct((NC, N, D), jnp.float32),
        mesh=mesh,
        scratch_types=[
            pltpu.VMEM((NCH, K), jnp.int32),
            pltpu.VMEM((NCH, K), jnp.int32),
            pltpu.VMEM((K, D), jnp.float32),
            pltpu.VMEM((ZR, D), jnp.float32),
            pltpu.VMEM_SHARED((N, D), jnp.float32),
            pltpu.SemaphoreType.DMA,
            pltpu.SemaphoreType.DMA,
        ],
        compiler_params=pltpu.CompilerParams(use_tc_tiling_on_sc=False),
    )


def _mm_body(x_ref, w_ref, o_ref):
    o_ref[...] = jnp.dot(x_ref[...], w_ref[...],
                         preferred_element_type=jnp.float32)


def _relu_mm_body(p_ref, w_ref, o_ref):
    r = jnp.maximum(p_ref[0] + p_ref[1], 0.0)
    o_ref[...] = jnp.dot(r, w_ref[...], preferred_element_type=jnp.float32)


def _add_body(q_ref, o_ref):
    o_ref[...] = q_ref[0] + q_ref[1]


@functools.lru_cache(maxsize=None)
def _layers():
    return _sc_edge_scatter(128), _sc_edge_scatter(64)


def kernel(x, adj, W1, W2):
    src = adj[0].astype(jnp.int32).reshape(NC, NS, NCH, K)
    dst = adj[1].astype(jnp.int32).reshape(NC, NS, NCH, K)
    sc1, sc2 = _layers()

    h1 = pl.pallas_call(
        _mm_body,
        out_shape=jax.ShapeDtypeStruct((N, 128), jnp.float32),
    )(x, W1)
    p1 = sc1(h1, src, dst)
    h2 = pl.pallas_call(
        _relu_mm_body,
        out_shape=jax.ShapeDtypeStruct((N, 64), jnp.float32),
    )(p1, W2)
    p2 = sc2(h2, src, dst)
    out = pl.pallas_call(
        _add_body,
        out_shape=jax.ShapeDtypeStruct((N, 64), jnp.float32),
    )(p2)
    return out
